# Initial kernel scaffold; baseline (speedup 1.0000x reference)
import jax
import jax.numpy as jnp
from jax.experimental import pallas as pl


def kernel(x, edge_index, W1, a_src1, a_dst1, b1, W2, a_src2, a_dst2, b2):
    raise NotImplementedError("write your pallas kernel here")


# speedup vs baseline: 40.5174x; 40.5174x over previous
import functools

import jax
import jax.numpy as jnp
from jax import lax
from jax.experimental import pallas as pl
from jax.experimental.pallas import tpu as pltpu
from jax.experimental.pallas import tpu_sc as plsc

N = 10000
E = 320000
D_IN = 128
H1 = 8
C1 = 8
D1 = H1 * C1
NC = 16

BLK = 256
N_PAD = 10240
GRID = N_PAD // BLK
DUMMY = N

NCORES = 2
NSUB = 16
NW = NCORES * NSUB
CHUNK = 128
CPW = (-(-E // (NW * CHUNK)) + 7) // 8 * 8
E_PAD = NW * CPW * CHUNK
N_CHUNKS = E_PAD // CHUNK

F32 = jnp.float32


def _leaky(v):
    return jnp.where(v > 0.0, v, v * 0.2)


def _take16(v, idx):
    dnums = lax.GatherDimensionNumbers(
        offset_dims=(), collapsed_slice_dims=(0,), start_index_map=(0,))
    return lax.gather(v, idx[:, None], dnums, (1,),
                      mode=lax.GatherScatterMode.PROMISE_IN_BOUNDS)


def _make_edge_kernel(d_t, heads8):
    mesh = plsc.VectorSubcoreMesh(
        core_axis_name="c", subcore_axis_name="s",
        num_cores=NCORES, num_subcores=NSUB)
    rows_per_tile = N_PAD // NSUB
    nz = rows_per_tile // CHUNK

    @functools.partial(
        pl.kernel, mesh=mesh,
        compiler_params=pltpu.CompilerParams(use_tc_tiling_on_sc=False),
        out_type=jax.ShapeDtypeStruct((NCORES, N_PAD, d_t), F32),
        scratch_types=[
            pltpu.VMEM_SHARED((N_PAD, d_t), F32),
            pltpu.VMEM((CPW, CHUNK), jnp.int32),
            pltpu.VMEM((CPW, CHUNK), jnp.int32),
            pltpu.VMEM((CHUNK, d_t), F32),
            pltpu.VMEM((CHUNK, 16), F32),
            pltpu.VMEM((CHUNK, d_t), F32),
            pltpu.SemaphoreType.DMA,
            pltpu.SemaphoreType.DMA,
        ])
    def ek(table, adt, srcp, dstp, out, acc, src_v, dst_v, rows_v, ad_v,
           msg_v, sem0, sem1):
        c = lax.axis_index("c")
        s = lax.axis_index("s")
        wid = s * NCORES + c
        tbase = s * rows_per_tile
        lane = lax.iota(jnp.int32, 16)

        def zrow(r, carry):
            for k in range(d_t // 16):
                msg_v[r, pl.ds(16 * k, 16)] = jnp.zeros((16,), F32)
            return carry
        lax.fori_loop(0, CHUNK, zrow, 0)
        for z in range(nz):
            pltpu.sync_copy(msg_v, acc.at[pl.ds(tbase + z * CHUNK, CHUNK)])
        plsc.subcore_barrier()

        pltpu.sync_copy(srcp.at[pl.ds(wid * CPW, CPW)], src_v)
        pltpu.sync_copy(dstp.at[pl.ds(wid * CPW, CPW)], dst_v)

        def chunk_body(j, carry):
            pltpu.async_copy(table.at[src_v.at[j]], rows_v, sem0).wait()
            pltpu.async_copy(adt.at[dst_v.at[j]], ad_v, sem1).wait()

            if heads8:
                def edge(e, cc):
                    asv = rows_v[e, pl.ds(64, 16)]
                    adv = ad_v[e, pl.ds(0, 16)]
                    w = jnp.exp(_leaky(asv + adv))
                    hi = jnp.where(lane >= 8, 1, 0)
                    for q in range(4):
                        wb = _take16(w, hi + 2 * q)
                        msg_v[e, pl.ds(16 * q, 16)] = (
                            rows_v[e, pl.ds(16 * q, 16)] * wb)
                    msg_v[e, pl.ds(64, 16)] = jnp.where(lane < 8, w, 0.0)
                    return cc
            else:
                def edge(e, cc):
                    asv = rows_v[e, pl.ds(16, 16)]
                    adv = ad_v[e, pl.ds(0, 16)]
                    w = jnp.exp(_leaky(asv + adv))
                    wb = _take16(w, jnp.zeros((16,), jnp.int32))
                    msg_v[e, pl.ds(0, 16)] = rows_v[e, pl.ds(0, 16)] * wb
                    msg_v[e, pl.ds(16, 16)] = jnp.where(lane < 1, w, 0.0)
                    return cc
            lax.fori_loop(0, CHUNK, edge, 0)

            pltpu.sync_copy(msg_v, acc.at[dst_v.at[j]], add=True)
            return carry
        lax.fori_loop(0, CPW, chunk_body, 0)

        plsc.subcore_barrier()
        for z in range(nz):
            pltpu.sync_copy(acc.at[pl.ds(tbase + z * CHUNK, CHUNK)], msg_v)
            pltpu.sync_copy(msg_v, out.at[c, pl.ds(tbase + z * CHUNK, CHUNK)])

    return ek


_edge1 = _make_edge_kernel(80, True)
_edge2 = _make_edge_kernel(32, False)


def _prep1_body(x_ref, w_ref, asm_ref, adm_ref, t_ref, ad_ref):
    h = jnp.dot(x_ref[...], w_ref[...], preferred_element_type=F32)
    als = jnp.dot(h, asm_ref[...], preferred_element_type=F32)
    ald = jnp.dot(h, adm_ref[...], preferred_element_type=F32)
    z8 = jnp.zeros((BLK, 8), F32)
    t_ref[...] = jnp.concatenate([h, als, z8], axis=1)
    ad_ref[...] = jnp.concatenate([ald, z8], axis=1)


def _mid_body(acc_ref, t_ref, ad_ref, w2_ref, as2_ref, ad2_ref, b1_ref,
              r_ref, t2_ref, ad2t_ref):
    acc = acc_ref[0] + acc_ref[1]
    h1 = t_ref[:, 0:64]
    sv = t_ref[:, 64:72] + ad_ref[:, 0:8]
    ws = jnp.exp(_leaky(sv))
    rmat = r_ref[...]
    num = acc[:, 0:64] + jnp.dot(ws, rmat, preferred_element_type=F32) * h1
    den = jnp.dot(acc[:, 64:72] + ws, rmat, preferred_element_type=F32)
    out1 = num / den + b1_ref[...]
    x2 = jnp.where(out1 > 0.0, out1, jnp.exp(out1) - 1.0)
    h2 = jnp.dot(x2, w2_ref[...], preferred_element_type=F32)
    as2 = jnp.sum(h2 * as2_ref[...], axis=1, keepdims=True)
    ad2 = jnp.sum(h2 * ad2_ref[...], axis=1, keepdims=True)
    z15 = jnp.zeros((BLK, 15), F32)
    t2_ref[...] = jnp.concatenate([h2, as2, z15], axis=1)
    ad2t_ref[...] = jnp.concatenate([ad2, z15], axis=1)


def _fin_body(acc_ref, t2_ref, ad2t_ref, b2_ref, o_ref):
    acc = acc_ref[0] + acc_ref[1]
    h2 = t2_ref[:, 0:16]
    sv = t2_ref[:, 16:17] + ad2t_ref[:, 0:1]
    ws = jnp.exp(_leaky(sv))
    num = acc[:, 0:16] + ws * h2
    den = acc[:, 16:17] + ws
    logits = num / den + b2_ref[...]
    m = jnp.max(logits, axis=1, keepdims=True)
    ex = jnp.exp(logits - m)
    lse = jnp.log(jnp.sum(ex, axis=1, keepdims=True)) + m
    o_ref[...] = logits - lse


def _full(shape):
    return pl.BlockSpec(shape, lambda i: (0,) * len(shape))


def kernel(x, edge_index, W1, a_src1, a_dst1, b1, W2, a_src2, a_dst2, b2):
    xp = jnp.pad(x, ((0, N_PAD - N), (0, 0)))
    sel = jnp.repeat(jnp.eye(H1, dtype=F32), C1, axis=0)
    asm = a_src1.reshape(-1, 1) * sel
    adm = a_dst1.reshape(-1, 1) * sel
    rmat = jnp.repeat(jnp.eye(H1, dtype=F32), C1, axis=1)
    pad_idx = jnp.full((E_PAD - E,), DUMMY, dtype=jnp.int32)
    srcp = jnp.concatenate([edge_index[0], pad_idx]).reshape(N_CHUNKS, CHUNK)
    dstp = jnp.concatenate([edge_index[1], pad_idx]).reshape(N_CHUNKS, CHUNK)

    table1, adt1 = pl.pallas_call(
        _prep1_body,
        grid=(GRID,),
        in_specs=[pl.BlockSpec((BLK, D_IN), lambda i: (i, 0)),
                  _full((D_IN, D1)), _full((D1, H1)), _full((D1, H1))],
        out_specs=[pl.BlockSpec((BLK, 80), lambda i: (i, 0)),
                   pl.BlockSpec((BLK, 16), lambda i: (i, 0))],
        out_shape=[jax.ShapeDtypeStruct((N_PAD, 80), F32),
                   jax.ShapeDtypeStruct((N_PAD, 16), F32)],
    )(xp, W1, asm, adm)

    acc1 = _edge1(table1, adt1, srcp, dstp)

    table2, adt2 = pl.pallas_call(
        _mid_body,
        grid=(GRID,),
        in_specs=[pl.BlockSpec((NCORES, BLK, 80), lambda i: (0, i, 0)),
                  pl.BlockSpec((BLK, 80), lambda i: (i, 0)),
                  pl.BlockSpec((BLK, 16), lambda i: (i, 0)),
                  _full((D1, NC)), _full((1, NC)), _full((1, NC)),
                  _full((1, D1)), _full((H1, D1))],
        out_specs=[pl.BlockSpec((BLK, 32), lambda i: (i, 0)),
                   pl.BlockSpec((BLK, 16), lambda i: (i, 0))],
        out_shape=[jax.ShapeDtypeStruct((N_PAD, 32), F32),
                   jax.ShapeDtypeStruct((N_PAD, 16), F32)],
    )(acc1, table1, adt1, W2, a_src2, a_dst2, b1.reshape(1, D1), rmat)

    acc2 = _edge2(table2, adt2, srcp, dstp)

    outp = pl.pallas_call(
        _fin_body,
        grid=(GRID,),
        in_specs=[pl.BlockSpec((NCORES, BLK, 32), lambda i: (0, i, 0)),
                  pl.BlockSpec((BLK, 32), lambda i: (i, 0)),
                  pl.BlockSpec((BLK, 16), lambda i: (i, 0)),
                  _full((1, NC))],
        out_specs=pl.BlockSpec((BLK, NC), lambda i: (i, 0)),
        out_shape=jax.ShapeDtypeStruct((N_PAD, NC), F32),
    )(acc2, table2, adt2, b2.reshape(1, NC))

    return outp[:N]


# speedup vs baseline: 64.8218x; 1.5999x over previous
import functools

import jax
import jax.numpy as jnp
from jax import lax
from jax.experimental import pallas as pl
from jax.experimental.pallas import tpu as pltpu
from jax.experimental.pallas import tpu_sc as plsc

N = 10000
E = 320000
D_IN = 128
H1 = 8
C1 = 8
D1 = H1 * C1
NC = 16

BLK = 256
N_PAD = 10240
GRID = N_PAD // BLK
DUMMY = N

NCORES = 2
NSUB = 16
NW = NCORES * NSUB
CHUNK = 128
CPW = (-(-E // (NW * CHUNK)) + 7) // 8 * 8
E_PAD = NW * CPW * CHUNK
N_CHUNKS = E_PAD // CHUNK

F32 = jnp.float32


def _leaky(v):
    return jnp.where(v > 0.0, v, v * 0.2)


def _take16(v, idx):
    dnums = lax.GatherDimensionNumbers(
        offset_dims=(), collapsed_slice_dims=(0,), start_index_map=(0,))
    return lax.gather(v, idx[:, None], dnums, (1,),
                      mode=lax.GatherScatterMode.PROMISE_IN_BOUNDS)


def _make_edge_kernel(d_t, heads8):
    mesh = plsc.VectorSubcoreMesh(
        core_axis_name="c", subcore_axis_name="s",
        num_cores=NCORES, num_subcores=NSUB)
    rows_per_tile = N_PAD // NSUB
    nz = rows_per_tile // CHUNK

    @functools.partial(
        pl.kernel, mesh=mesh,
        compiler_params=pltpu.CompilerParams(use_tc_tiling_on_sc=False),
        out_type=jax.ShapeDtypeStruct((NCORES, N_PAD, d_t), F32),
        scratch_types=[
            pltpu.VMEM_SHARED((N_PAD, d_t), F32),
            pltpu.VMEM((CPW + 2, CHUNK), jnp.int32),
            pltpu.VMEM((CPW + 2, CHUNK), jnp.int32),
            pltpu.VMEM((CHUNK, d_t), F32),
            pltpu.VMEM((CHUNK, d_t), F32),
            pltpu.VMEM((CHUNK, 16), F32),
            pltpu.VMEM((CHUNK, 16), F32),
            pltpu.VMEM((CHUNK, d_t), F32),
            pltpu.VMEM((CHUNK, d_t), F32),
            pltpu.SemaphoreType.DMA, pltpu.SemaphoreType.DMA,
            pltpu.SemaphoreType.DMA, pltpu.SemaphoreType.DMA,
            pltpu.SemaphoreType.DMA, pltpu.SemaphoreType.DMA,
        ])
    def ek(table, adt, srcp, dstp, out, acc, src_v, dst_v, rows0, rows1,
           ad0, ad1, msg0, msg1, st0, st1, sa0, sa1, ss0, ss1):
        c = lax.axis_index("c")
        s = lax.axis_index("s")
        wid = s * NCORES + c
        tbase = s * rows_per_tile
        lane = lax.iota(jnp.int32, 16)
        rows = (rows0, rows1)
        ads = (ad0, ad1)
        msgs = (msg0, msg1)
        st = (st0, st1)
        sa = (sa0, sa1)
        ss = (ss0, ss1)

        def zrow(r, carry):
            for k in range(d_t // 16):
                msg0[r, pl.ds(16 * k, 16)] = jnp.zeros((16,), F32)
            return carry
        lax.fori_loop(0, CHUNK, zrow, 0)
        for z in range(nz):
            pltpu.sync_copy(msg0, acc.at[pl.ds(tbase + z * CHUNK, CHUNK)])
        plsc.subcore_barrier()

        pltpu.sync_copy(srcp.at[pl.ds(wid * CPW, CPW)], src_v.at[pl.ds(0, CPW)])
        pltpu.sync_copy(dstp.at[pl.ds(wid * CPW, CPW)], dst_v.at[pl.ds(0, CPW)])
        for r in (CPW, CPW + 1):
            for k in range(CHUNK // 16):
                src_v[r, pl.ds(16 * k, 16)] = jnp.zeros((16,), jnp.int32)
                dst_v[r, pl.ds(16 * k, 16)] = jnp.zeros((16,), jnp.int32)

        def compute_chunk(rows_b, ad_b, msg_b):
            if heads8:
                @plsc.parallel_loop(0, CHUNK, unroll=4)
                def edge(e):
                    asv = rows_b[e, pl.ds(64, 16)]
                    adv = ad_b[e, pl.ds(0, 16)]
                    w = jnp.exp(_leaky(asv + adv))
                    hi = jnp.where(lane >= 8, 1, 0)
                    for q in range(4):
                        wb = _take16(w, hi + 2 * q)
                        msg_b[e, pl.ds(16 * q, 16)] = (
                            rows_b[e, pl.ds(16 * q, 16)] * wb)
                    msg_b[e, pl.ds(64, 16)] = jnp.where(lane < 8, w, 0.0)
            else:
                @plsc.parallel_loop(0, CHUNK, unroll=4)
                def edge(e):
                    asv = rows_b[e, pl.ds(16, 16)]
                    adv = ad_b[e, pl.ds(0, 16)]
                    w = jnp.exp(_leaky(asv + adv))
                    wb = _take16(w, jnp.zeros((16,), jnp.int32))
                    msg_b[e, pl.ds(0, 16)] = rows_b[e, pl.ds(0, 16)] * wb
                    msg_b[e, pl.ds(16, 16)] = jnp.where(lane < 1, w, 0.0)

        for b in (0, 1):
            pltpu.async_copy(table.at[src_v.at[b]], rows[b], st[b])
            pltpu.async_copy(adt.at[dst_v.at[b]], ads[b], sa[b])

        def pair(j2, carry):
            for b in (0, 1):
                jj = 2 * j2 + b
                pltpu.make_async_copy(table.at[src_v.at[jj]], rows[b],
                                      st[b]).wait()
                pltpu.make_async_copy(adt.at[dst_v.at[jj]], ads[b],
                                      sa[b]).wait()

                @pl.when(j2 > 0)
                def _drain_prev_scatter():
                    pltpu.make_async_copy(msgs[b], acc.at[dst_v.at[jj - 2]],
                                          ss[b]).wait()

                compute_chunk(rows[b], ads[b], msgs[b])
                pltpu.async_copy(table.at[src_v.at[jj + 2]], rows[b], st[b])
                pltpu.async_copy(adt.at[dst_v.at[jj + 2]], ads[b], sa[b])
                pltpu.async_copy(msgs[b], acc.at[dst_v.at[jj]], ss[b],
                                 add=True)
            return carry
        lax.fori_loop(0, CPW // 2, pair, 0)

        for b in (0, 1):
            pltpu.make_async_copy(table.at[src_v.at[CPW + b]], rows[b],
                                  st[b]).wait()
            pltpu.make_async_copy(adt.at[dst_v.at[CPW + b]], ads[b],
                                  sa[b]).wait()
            pltpu.make_async_copy(msgs[b], acc.at[dst_v.at[CPW - 2 + b]],
                                  ss[b]).wait()

        plsc.subcore_barrier()
        for z in range(nz):
            pltpu.sync_copy(acc.at[pl.ds(tbase + z * CHUNK, CHUNK)], msg0)
            pltpu.sync_copy(msg0, out.at[c, pl.ds(tbase + z * CHUNK, CHUNK)])

    return ek


_edge1 = _make_edge_kernel(80, True)
_edge2 = _make_edge_kernel(32, False)


def _prep1_body(x_ref, w_ref, asm_ref, adm_ref, t_ref, ad_ref):
    h = jnp.dot(x_ref[...], w_ref[...], preferred_element_type=F32)
    als = jnp.dot(h, asm_ref[...], preferred_element_type=F32)
    ald = jnp.dot(h, adm_ref[...], preferred_element_type=F32)
    z8 = jnp.zeros((BLK, 8), F32)
    t_ref[...] = jnp.concatenate([h, als, z8], axis=1)
    ad_ref[...] = jnp.concatenate([ald, z8], axis=1)


def _mid_body(acc_ref, t_ref, ad_ref, w2_ref, as2_ref, ad2_ref, b1_ref,
              r_ref, t2_ref, ad2t_ref):
    acc = acc_ref[0] + acc_ref[1]
    h1 = t_ref[:, 0:64]
    sv = t_ref[:, 64:72] + ad_ref[:, 0:8]
    ws = jnp.exp(_leaky(sv))
    rmat = r_ref[...]
    num = acc[:, 0:64] + jnp.dot(ws, rmat, preferred_element_type=F32) * h1
    den = jnp.dot(acc[:, 64:72] + ws, rmat, preferred_element_type=F32)
    out1 = num / den + b1_ref[...]
    x2 = jnp.where(out1 > 0.0, out1, jnp.exp(out1) - 1.0)
    h2 = jnp.dot(x2, w2_ref[...], preferred_element_type=F32)
    as2 = jnp.sum(h2 * as2_ref[...], axis=1, keepdims=True)
    ad2 = jnp.sum(h2 * ad2_ref[...], axis=1, keepdims=True)
    z15 = jnp.zeros((BLK, 15), F32)
    t2_ref[...] = jnp.concatenate([h2, as2, z15], axis=1)
    ad2t_ref[...] = jnp.concatenate([ad2, z15], axis=1)


def _fin_body(acc_ref, t2_ref, ad2t_ref, b2_ref, o_ref):
    acc = acc_ref[0] + acc_ref[1]
    h2 = t2_ref[:, 0:16]
    sv = t2_ref[:, 16:17] + ad2t_ref[:, 0:1]
    ws = jnp.exp(_leaky(sv))
    num = acc[:, 0:16] + ws * h2
    den = acc[:, 16:17] + ws
    logits = num / den + b2_ref[...]
    m = jnp.max(logits, axis=1, keepdims=True)
    ex = jnp.exp(logits - m)
    lse = jnp.log(jnp.sum(ex, axis=1, keepdims=True)) + m
    o_ref[...] = logits - lse


def _full(shape):
    return pl.BlockSpec(shape, lambda i: (0,) * len(shape))


def kernel(x, edge_index, W1, a_src1, a_dst1, b1, W2, a_src2, a_dst2, b2):
    xp = jnp.pad(x, ((0, N_PAD - N), (0, 0)))
    sel = jnp.repeat(jnp.eye(H1, dtype=F32), C1, axis=0)
    asm = a_src1.reshape(-1, 1) * sel
    adm = a_dst1.reshape(-1, 1) * sel
    rmat = jnp.repeat(jnp.eye(H1, dtype=F32), C1, axis=1)
    pad_idx = jnp.full((E_PAD - E,), DUMMY, dtype=jnp.int32)
    srcp = jnp.concatenate([edge_index[0], pad_idx]).reshape(N_CHUNKS, CHUNK)
    dstp = jnp.concatenate([edge_index[1], pad_idx]).reshape(N_CHUNKS, CHUNK)

    table1, adt1 = pl.pallas_call(
        _prep1_body,
        grid=(GRID,),
        in_specs=[pl.BlockSpec((BLK, D_IN), lambda i: (i, 0)),
                  _full((D_IN, D1)), _full((D1, H1)), _full((D1, H1))],
        out_specs=[pl.BlockSpec((BLK, 80), lambda i: (i, 0)),
                   pl.BlockSpec((BLK, 16), lambda i: (i, 0))],
        out_shape=[jax.ShapeDtypeStruct((N_PAD, 80), F32),
                   jax.ShapeDtypeStruct((N_PAD, 16), F32)],
    )(xp, W1, asm, adm)

    acc1 = _edge1(table1, adt1, srcp, dstp)

    table2, adt2 = pl.pallas_call(
        _mid_body,
        grid=(GRID,),
        in_specs=[pl.BlockSpec((NCORES, BLK, 80), lambda i: (0, i, 0)),
                  pl.BlockSpec((BLK, 80), lambda i: (i, 0)),
                  pl.BlockSpec((BLK, 16), lambda i: (i, 0)),
                  _full((D1, NC)), _full((1, NC)), _full((1, NC)),
                  _full((1, D1)), _full((H1, D1))],
        out_specs=[pl.BlockSpec((BLK, 32), lambda i: (i, 0)),
                   pl.BlockSpec((BLK, 16), lambda i: (i, 0))],
        out_shape=[jax.ShapeDtypeStruct((N_PAD, 32), F32),
                   jax.ShapeDtypeStruct((N_PAD, 16), F32)],
    )(acc1, table1, adt1, W2, a_src2, a_dst2, b1.reshape(1, D1), rmat)

    acc2 = _edge2(table2, adt2, srcp, dstp)

    outp = pl.pallas_call(
        _fin_body,
        grid=(GRID,),
        in_specs=[pl.BlockSpec((NCORES, BLK, 32), lambda i: (0, i, 0)),
                  pl.BlockSpec((BLK, 32), lambda i: (i, 0)),
                  pl.BlockSpec((BLK, 16), lambda i: (i, 0)),
                  _full((1, NC))],
        out_specs=pl.BlockSpec((BLK, NC), lambda i: (i, 0)),
        out_shape=jax.ShapeDtypeStruct((N_PAD, NC), F32),
    )(acc2, table2, adt2, b2.reshape(1, NC))

    return outp[:N]
